# Initial kernel scaffold; baseline (speedup 1.0000x reference)
import jax
import jax.numpy as jnp
from jax.experimental import pallas as pl


def kernel(x, edge_index, Gdatax, Gdataedge_index, batch, index, W_e1, b_e1, W_e2, b_e2, mW1, mb1, mW2, mb2, mW3, mb3):
    raise NotImplementedError("write your pallas kernel here")


# speedup vs baseline: 1.0102x; 1.0102x over previous
import functools
import jax
import jax.numpy as jnp
from jax.experimental import pallas as pl
from jax.experimental.pallas import tpu as pltpu

N_NODES = 10000
D = 256
N_GRAPHS = 64
ROW_BLK = 1000
POOL_PAD = 10240
POOL_BLK = 2048


def _mish(v):
    return v * jnp.tanh(jax.nn.softplus(v))


def _enc_matmul_body(x_ref, w_ref, o_ref):
    o_ref[...] = jnp.dot(x_ref[0], w_ref[0],
                         preferred_element_type=jnp.float32)[None]


def _enc_matmul(X2, W2):
    n = X2.shape[1]
    grid = (2, n // ROW_BLK)
    return pl.pallas_call(
        _enc_matmul_body,
        grid=grid,
        in_specs=[
            pl.BlockSpec((1, ROW_BLK, D), lambda b, i: (b, i, 0)),
            pl.BlockSpec((1, D, D), lambda b, i: (b, 0, 0)),
        ],
        out_specs=pl.BlockSpec((1, ROW_BLK, D), lambda b, i: (b, i, 0)),
        out_shape=jax.ShapeDtypeStruct((2, n, D), jnp.float32),
    )(X2, W2)


def _pool_body(h_ref, batch_ref, idx_ref, bias_ref,
               z_ref, sum_ref, cnt_ref):
    g = pl.program_id(0)
    nblk = pl.num_programs(0)

    @pl.when(g == 0)
    def _init():
        z_ref[...] = jnp.zeros_like(z_ref)
        sum_ref[...] = jnp.zeros_like(sum_ref)
        cnt_ref[...] = jnp.zeros_like(cnt_ref)

    h = h_ref[...] + bias_ref[...]
    bb = batch_ref[0]
    rows64 = jax.lax.broadcasted_iota(jnp.int32, (N_GRAPHS, POOL_BLK), 0)
    onehot_b = (bb == rows64).astype(jnp.float32)
    cols = jax.lax.broadcasted_iota(jnp.int32, (N_GRAPHS, POOL_BLK), 1)
    cols = cols + g * POOL_BLK
    onehot_z = (idx_ref[...] == cols).astype(jnp.float32)

    sum_ref[...] += jnp.dot(onehot_b, h, preferred_element_type=jnp.float32)
    z_ref[...] += jnp.dot(onehot_z, h, preferred_element_type=jnp.float32)
    cnt_ref[...] += jnp.sum(onehot_b, axis=1, keepdims=True)

    @pl.when(g == nblk - 1)
    def _fin():
        sum_ref[...] = sum_ref[...] / jnp.maximum(cnt_ref[...], 1.0)


def _pool_and_gather(hidden_pad, batch_pad, index, bias):
    nblk = POOL_PAD // POOL_BLK
    z, summary, _cnt = pl.pallas_call(
        _pool_body,
        grid=(nblk,),
        in_specs=[
            pl.BlockSpec((POOL_BLK, D), lambda g: (g, 0)),
            pl.BlockSpec((1, 1, POOL_BLK), lambda g: (g, 0, 0)),
            pl.BlockSpec((N_GRAPHS, 1), lambda g: (0, 0)),
            pl.BlockSpec((1, D), lambda g: (0, 0)),
        ],
        out_specs=[
            pl.BlockSpec((N_GRAPHS, D), lambda g: (0, 0)),
            pl.BlockSpec((N_GRAPHS, D), lambda g: (0, 0)),
            pl.BlockSpec((N_GRAPHS, 1), lambda g: (0, 0)),
        ],
        out_shape=[
            jax.ShapeDtypeStruct((N_GRAPHS, D), jnp.float32),
            jax.ShapeDtypeStruct((N_GRAPHS, D), jnp.float32),
            jax.ShapeDtypeStruct((N_GRAPHS, 1), jnp.float32),
        ],
    )(hidden_pad, batch_pad, index, bias)
    return z, summary


def _mlp_body(x_ref, badd_ref, w1_ref, b1_ref, w2_ref, b2_ref,
              w3_ref, b3_ref, o_ref):
    x = x_ref[...] + badd_ref[...]
    f = _mish(jnp.dot(x, w1_ref[...], preferred_element_type=jnp.float32)
              + b1_ref[...])
    f = _mish(jnp.dot(f, w2_ref[...], preferred_element_type=jnp.float32)
              + b2_ref[...])
    f1 = _mish(jnp.dot(x, w3_ref[...], preferred_element_type=jnp.float32)
               + b3_ref[...])
    o_ref[...] = f + f1 + x


def _mlp(X, badd, mW1, mb1, mW2, mb2, mW3, mb3, blk):
    n = X.shape[0]
    grid = (n // blk,)
    full = lambda g: (0, 0)
    return pl.pallas_call(
        _mlp_body,
        grid=grid,
        in_specs=[
            pl.BlockSpec((blk, D), lambda g: (g, 0)),
            pl.BlockSpec((1, D), full),
            pl.BlockSpec((D, D), full), pl.BlockSpec((1, D), full),
            pl.BlockSpec((D, D), full), pl.BlockSpec((1, D), full),
            pl.BlockSpec((D, D), full), pl.BlockSpec((1, D), full),
        ],
        out_specs=pl.BlockSpec((blk, D), lambda g: (g, 0)),
        out_shape=jax.ShapeDtypeStruct((n, D), jnp.float32),
    )(X, badd, mW1, mb1, mW2, mb2, mW3, mb3)


def _conv_segments(h, edge_index):
    src = edge_index[0]
    dst = edge_index[1]
    ones = jnp.ones((edge_index.shape[1],), dtype=h.dtype)
    deg = jax.ops.segment_sum(ones, dst, num_segments=N_NODES)
    deg = jnp.clip(deg, 1.0, None)
    inv_sqrt = 1.0 / jnp.sqrt(deg)
    norm = inv_sqrt[src] * inv_sqrt[dst]
    msgs = h[src] * norm[:, None]
    return jax.ops.segment_sum(msgs, dst, num_segments=N_NODES)


def kernel(x, edge_index, Gdatax, Gdataedge_index, batch, index,
           W_e1, b_e1, W_e2, b_e2, mW1, mb1, mW2, mb2, mW3, mb3):
    X2 = jnp.stack([x, Gdatax])
    W2 = jnp.stack([W_e1, W_e2])
    H = _enc_matmul(X2, W2)

    conv1 = _conv_segments(H[0], edge_index)
    conv2 = _conv_segments(H[1], Gdataedge_index)

    hidden_pad = jnp.pad(conv1, ((0, POOL_PAD - N_NODES), (0, 0)))
    batch_pad = jnp.pad(batch.astype(jnp.int32),
                        (0, POOL_PAD - N_NODES),
                        constant_values=jnp.int32(1 << 30))
    batch_pad = batch_pad.reshape(POOL_PAD // POOL_BLK, 1, POOL_BLK)
    z0, summary0 = _pool_and_gather(hidden_pad, batch_pad,
                                    index.astype(jnp.int32).reshape(N_GRAPHS, 1),
                                    b_e1.reshape(1, D))

    mb1r, mb2r, mb3r = (b.reshape(1, D) for b in (mb1, mb2, mb3))
    Goutput = _mlp(conv2, b_e2.reshape(1, D), mW1, mb1r, mW2, mb2r,
                   mW3, mb3r, ROW_BLK)
    zs = jnp.concatenate([z0, summary0], axis=0)
    zs_out = _mlp(zs, jnp.zeros((1, D), jnp.float32), mW1, mb1r,
                  mW2, mb2r, mW3, mb3r, 2 * N_GRAPHS)
    return zs_out[:N_GRAPHS], zs_out[N_GRAPHS:], Goutput


# speedup vs baseline: 7.4477x; 7.3727x over previous
import functools
import jax
import jax.numpy as jnp
from jax import lax
from jax.experimental import pallas as pl
from jax.experimental.pallas import tpu as pltpu
from jax.experimental.pallas import tpu_sc as plsc

N_NODES = 10000
D = 256
HALF = 128
N_GRAPHS = 64

N_PAD = 10240
E = 160000
K = 128
NCHUNK = 80
GC = 16
NGRP = NCHUNK // GC
E_PAD = 16 * NCHUNK * K
EPW = E_PAD // 32
ROWS_PT = N_PAD // 16

ROW_BLK = 1024
POOL_BLK = 2048
PREP_BLK = 512

_sc_mesh = plsc.VectorSubcoreMesh(core_axis_name="c", subcore_axis_name="s")


def _mish(v):
    return v * jnp.tanh(jax.nn.softplus(v))


def _deg_one(dref, out_hbm, g, w, dbuf, hist):
    @pl.loop(0, N_PAD // 16)
    def _zero(i):
        hist[pl.ds(i * 16, 16)] = jnp.zeros((16,), jnp.float32)

    pltpu.sync_copy(dref.at[w], dbuf)
    ones16 = jnp.ones((16,), jnp.float32)

    @pl.loop(0, EPW // 16)
    def _acc(i):
        idx = dbuf[pl.ds(i * 16, 16)]
        plsc.addupdate_scatter(hist, [idx], ones16)

    pltpu.sync_copy(hist, out_hbm.at[g, w])


@functools.partial(
    pl.kernel,
    out_type=jax.ShapeDtypeStruct((2, 32, N_PAD), jnp.float32),
    mesh=_sc_mesh,
    compiler_params=pltpu.CompilerParams(needs_layout_passes=False),
    scratch_types=[
        pltpu.VMEM((EPW,), jnp.int32),
        pltpu.VMEM((N_PAD,), jnp.float32),
    ],
)
def _deg_body(d1, d2, out_hbm, dbuf, hist):
    c = lax.axis_index("c")
    s = lax.axis_index("s")
    w = c * 16 + s
    _deg_one(d1, out_hbm, 0, w, dbuf, hist)
    _deg_one(d2, out_hbm, 1, w, dbuf, hist)


def _conv_half(h, srcr, dstr, outr, s, acc, rbuf, gbuf, gsem0, gsem1, isem):
    @pl.loop(0, K)
    def _zz(i):
        for j in range(HALF // 16):
            rbuf[0, i, pl.ds(j * 16, 16)] = jnp.zeros((16,), jnp.float32)

    for k in range(ROWS_PT // K):
        pltpu.sync_copy(rbuf.at[0], acc.at[pl.ds(s * ROWS_PT + k * K, K)])

    def load_group(g, slot):
        pltpu.async_copy(srcr.at[s, pl.ds(g * GC, GC)], gbuf.at[slot, 0],
                         isem)
        pltpu.async_copy(dstr.at[s, pl.ds(g * GC, GC)], gbuf.at[slot, 1],
                         isem)

    def wait_group():
        pltpu.make_async_copy(srcr.at[s, pl.ds(0, GC)], gbuf.at[0, 0],
                              isem).wait()
        pltpu.make_async_copy(dstr.at[s, pl.ds(0, GC)], gbuf.at[0, 1],
                              isem).wait()

    def wait_rows(b, sem):
        pltpu.make_async_copy(h.at[gbuf.at[0, 0, 0]], rbuf.at[b], sem).wait()

    load_group(0, 0)
    plsc.subcore_barrier()

    @pl.loop(0, NGRP)
    def _grp(g):
        b = lax.rem(g, 2)
        wait_group()
        load_group(lax.rem(g + 1, NGRP), 1 - b)
        pltpu.async_copy(h.at[gbuf.at[b, 0, 0]], rbuf.at[0], gsem0)
        for k in range(1, GC):
            bb = k % 2
            sem = gsem1 if bb else gsem0
            psem = gsem0 if bb else gsem1
            pltpu.async_copy(h.at[gbuf.at[b, 0, k]], rbuf.at[bb], sem)
            wait_rows(1 - bb, psem)
            pltpu.sync_copy(rbuf.at[1 - bb], acc.at[gbuf.at[b, 1, k - 1]],
                            add=True)
        wait_rows((GC - 1) % 2, gsem1 if (GC - 1) % 2 else gsem0)
        pltpu.sync_copy(rbuf.at[(GC - 1) % 2], acc.at[gbuf.at[b, 1, GC - 1]],
                        add=True)

    wait_group()
    plsc.subcore_barrier()

    for k in range(ROWS_PT // K):
        r0 = s * ROWS_PT + k * K
        pltpu.sync_copy(acc.at[pl.ds(r0, K)], rbuf.at[0])
        pltpu.sync_copy(rbuf.at[0], outr.at[pl.ds(r0, K)])
    plsc.subcore_barrier()


@functools.partial(
    pl.kernel,
    out_type=[jax.ShapeDtypeStruct((N_PAD, HALF), jnp.float32)] * 4,
    mesh=_sc_mesh,
    compiler_params=pltpu.CompilerParams(needs_layout_passes=False),
    scratch_types=[
        pltpu.VMEM_SHARED((N_PAD, HALF), jnp.float32),
        pltpu.VMEM((2, K, HALF), jnp.float32),
        pltpu.VMEM((2, 2, GC, K), jnp.int32),
        pltpu.SemaphoreType.DMA,
        pltpu.SemaphoreType.DMA,
        pltpu.SemaphoreType.DMA,
    ],
)
def _conv_body(h1a, h1b, h2a, h2b, esrc, edst, gsrc, gdst,
               o1a, o1b, o2a, o2b,
               acc, rbuf, gbuf, gsem0, gsem1, isem):
    c = lax.axis_index("c")
    s = lax.axis_index("s")

    @pl.when(c == 0)
    def _c1a():
        _conv_half(h1a, esrc, edst, o1a, s, acc, rbuf, gbuf,
                   gsem0, gsem1, isem)

    @pl.when(c == 1)
    def _c1b():
        _conv_half(h1b, esrc, edst, o1b, s, acc, rbuf, gbuf,
                   gsem0, gsem1, isem)

    @pl.when(c == 0)
    def _c2a():
        _conv_half(h2a, gsrc, gdst, o2a, s, acc, rbuf, gbuf,
                   gsem0, gsem1, isem)

    @pl.when(c == 1)
    def _c2b():
        _conv_half(h2b, gsrc, gdst, o2b, s, acc, rbuf, gbuf,
                   gsem0, gsem1, isem)


def _prep_body(deg_ref, o_ref):
    sm = jnp.sum(deg_ref[0], axis=0, keepdims=True)
    isq = jax.lax.rsqrt(jnp.maximum(sm, 1.0))
    r = lax.broadcasted_iota(jnp.int32, (PREP_BLK, PREP_BLK), 0)
    col = lax.broadcasted_iota(jnp.int32, (PREP_BLK, PREP_BLK), 1)
    o_ref[0] = jnp.sum(jnp.where(r == col, isq, 0.0), axis=1, keepdims=True)


def _prep(deg_part):
    return pl.pallas_call(
        _prep_body,
        grid=(2, N_PAD // PREP_BLK),
        in_specs=[pl.BlockSpec((1, 32, PREP_BLK), lambda b, i: (b, 0, i))],
        out_specs=pl.BlockSpec((1, PREP_BLK, 1), lambda b, i: (b, i, 0)),
        out_shape=jax.ShapeDtypeStruct((2, N_PAD, 1), jnp.float32),
    )(deg_part)


def _enc_body(x_ref, isq_ref, w_ref, o_ref):
    xs = x_ref[0] * isq_ref[0]
    o_ref[0, 0] = jnp.dot(xs, w_ref[0], preferred_element_type=jnp.float32)


def _enc(X2, isq2, W2):
    return pl.pallas_call(
        _enc_body,
        grid=(2, 2, N_PAD // ROW_BLK),
        in_specs=[
            pl.BlockSpec((1, ROW_BLK, D), lambda b, h, i: (b, i, 0)),
            pl.BlockSpec((1, ROW_BLK, 1), lambda b, h, i: (b, i, 0)),
            pl.BlockSpec((1, D, HALF), lambda b, h, i: (b, 0, h)),
        ],
        out_specs=pl.BlockSpec((1, 1, ROW_BLK, HALF),
                               lambda b, h, i: (b, h, i, 0)),
        out_shape=jax.ShapeDtypeStruct((2, 2, N_PAD, HALF), jnp.float32),
    )(X2, isq2, W2)


def _pool_body(a_ref, b_ref, isq_ref, batch_ref, idx_ref, bias_ref,
               z_ref, sum_ref, cnt_ref):
    g = pl.program_id(0)
    nblk = pl.num_programs(0)

    @pl.when(g == 0)
    def _init():
        z_ref[...] = jnp.zeros_like(z_ref)
        sum_ref[...] = jnp.zeros_like(sum_ref)
        cnt_ref[...] = jnp.zeros_like(cnt_ref)

    h = jnp.concatenate([a_ref[...], b_ref[...]], axis=1)
    h = h * isq_ref[...] + bias_ref[...]
    bb = batch_ref[0]
    rows64 = lax.broadcasted_iota(jnp.int32, (N_GRAPHS, POOL_BLK), 0)
    onehot_b = (bb == rows64).astype(jnp.float32)
    cols = lax.broadcasted_iota(jnp.int32, (N_GRAPHS, POOL_BLK), 1)
    cols = cols + g * POOL_BLK
    onehot_z = (idx_ref[...] == cols).astype(jnp.float32)

    sum_ref[...] += jnp.dot(onehot_b, h, preferred_element_type=jnp.float32)
    z_ref[...] += jnp.dot(onehot_z, h, preferred_element_type=jnp.float32)
    cnt_ref[...] += jnp.sum(onehot_b, axis=1, keepdims=True)

    @pl.when(g == nblk - 1)
    def _fin():
        sum_ref[...] = sum_ref[...] / jnp.maximum(cnt_ref[...], 1.0)


def _pool_and_gather(o1a, o1b, isq1, batch3, index, bias):
    nblk = N_PAD // POOL_BLK
    z, summary, _cnt = pl.pallas_call(
        _pool_body,
        grid=(nblk,),
        in_specs=[
            pl.BlockSpec((POOL_BLK, HALF), lambda g: (g, 0)),
            pl.BlockSpec((POOL_BLK, HALF), lambda g: (g, 0)),
            pl.BlockSpec((POOL_BLK, 1), lambda g: (g, 0)),
            pl.BlockSpec((1, 1, POOL_BLK), lambda g: (g, 0, 0)),
            pl.BlockSpec((N_GRAPHS, 1), lambda g: (0, 0)),
            pl.BlockSpec((1, D), lambda g: (0, 0)),
        ],
        out_specs=[
            pl.BlockSpec((N_GRAPHS, D), lambda g: (0, 0)),
            pl.BlockSpec((N_GRAPHS, D), lambda g: (0, 0)),
            pl.BlockSpec((N_GRAPHS, 1), lambda g: (0, 0)),
        ],
        out_shape=[
            jax.ShapeDtypeStruct((N_GRAPHS, D), jnp.float32),
            jax.ShapeDtypeStruct((N_GRAPHS, D), jnp.float32),
            jax.ShapeDtypeStruct((N_GRAPHS, 1), jnp.float32),
        ],
    )(o1a, o1b, isq1, batch3, index, bias)
    return z, summary


def _mlp_body(a_ref, b_ref, isq_ref, badd_ref, w1_ref, b1_ref,
              w2_ref, b2_ref, w3_ref, b3_ref, o_ref):
    x = jnp.concatenate([a_ref[...], b_ref[...]], axis=1)
    x = x * isq_ref[...] + badd_ref[...]
    f = _mish(jnp.dot(x, w1_ref[...], preferred_element_type=jnp.float32)
              + b1_ref[...])
    f = _mish(jnp.dot(f, w2_ref[...], preferred_element_type=jnp.float32)
              + b2_ref[...])
    f1 = _mish(jnp.dot(x, w3_ref[...], preferred_element_type=jnp.float32)
               + b3_ref[...])
    o_ref[...] = f + f1 + x


def _mlp(Xa, Xb, isq, badd, mW1, mb1, mW2, mb2, mW3, mb3, blk):
    n = Xa.shape[0]
    full = lambda g: (0, 0)
    return pl.pallas_call(
        _mlp_body,
        grid=(n // blk,),
        in_specs=[
            pl.BlockSpec((blk, HALF), lambda g: (g, 0)),
            pl.BlockSpec((blk, HALF), lambda g: (g, 0)),
            pl.BlockSpec((blk, 1), lambda g: (g, 0)),
            pl.BlockSpec((1, D), full),
            pl.BlockSpec((D, D), full), pl.BlockSpec((1, D), full),
            pl.BlockSpec((D, D), full), pl.BlockSpec((1, D), full),
            pl.BlockSpec((D, D), full), pl.BlockSpec((1, D), full),
        ],
        out_specs=pl.BlockSpec((blk, D), lambda g: (g, 0)),
        out_shape=jax.ShapeDtypeStruct((n, D), jnp.float32),
    )(Xa, Xb, isq, badd, mW1, mb1, mW2, mb2, mW3, mb3)


def _pad_edges(edge_index):
    src = edge_index[0].astype(jnp.int32)
    dst = edge_index[1].astype(jnp.int32)
    nfill = E_PAD - E
    src = jnp.concatenate([src, jnp.zeros((nfill,), jnp.int32)])
    dst = jnp.concatenate([dst, jnp.full((nfill,), N_PAD - 1, jnp.int32)])
    return (src.reshape(16, NCHUNK, K), dst.reshape(16, NCHUNK, K),
            dst.reshape(32, EPW))


def kernel(x, edge_index, Gdatax, Gdataedge_index, batch, index,
           W_e1, b_e1, W_e2, b_e2, mW1, mb1, mW2, mb2, mW3, mb3):
    src16_1, dst16_1, dst32_1 = _pad_edges(edge_index)
    src16_2, dst16_2, dst32_2 = _pad_edges(Gdataedge_index)

    deg_part = _deg_body(dst32_1, dst32_2)
    isq2 = _prep(deg_part)

    pad = ((0, N_PAD - N_NODES), (0, 0))
    X2 = jnp.stack([jnp.pad(x, pad), jnp.pad(Gdatax, pad)])
    W2 = jnp.stack([W_e1, W_e2])
    Hh = _enc(X2, isq2, W2)

    o1a, o1b, o2a, o2b = _conv_body(
        Hh[0, 0], Hh[0, 1], Hh[1, 0], Hh[1, 1],
        src16_1, dst16_1, src16_2, dst16_2)

    batch_pad = jnp.pad(batch.astype(jnp.int32), (0, N_PAD - N_NODES),
                        constant_values=jnp.int32(1 << 30))
    batch3 = batch_pad.reshape(N_PAD // POOL_BLK, 1, POOL_BLK)
    z0, summary0 = _pool_and_gather(
        o1a, o1b, isq2[0], batch3,
        index.astype(jnp.int32).reshape(N_GRAPHS, 1), b_e1.reshape(1, D))

    mb1r, mb2r, mb3r = (b.reshape(1, D) for b in (mb1, mb2, mb3))
    Goutput = _mlp(o2a, o2b, isq2[1], b_e2.reshape(1, D),
                   mW1, mb1r, mW2, mb2r, mW3, mb3r, ROW_BLK)[:N_NODES]

    zs = jnp.concatenate([z0, summary0], axis=0)
    ones_small = jnp.ones((2 * N_GRAPHS, 1), jnp.float32)
    zs_out = _mlp(zs[:, :HALF], zs[:, HALF:], ones_small,
                  jnp.zeros((1, D), jnp.float32),
                  mW1, mb1r, mW2, mb2r, mW3, mb3r, 2 * N_GRAPHS)
    return zs_out[:N_GRAPHS], zs_out[N_GRAPHS:], Goutput
